# baseline (speedup 1.0000x reference)
import functools

import jax
import jax.numpy as jnp
from jax import lax
from jax.experimental import pallas as pl
from jax.experimental.pallas import tpu as pltpu
from jax.experimental.pallas import tpu_sc as plsc

VOCAB = 100000
EMBED_DIM = 16
BATCH = 1024

_NC = 2
_NS = 16
_NW = _NC * _NS
_B_PER_W = BATCH // _NW

_TILE_M = 5120


@functools.partial(
    pl.kernel,
    mesh=plsc.VectorSubcoreMesh(core_axis_name="c", subcore_axis_name="s"),
    out_type=jax.ShapeDtypeStruct((BATCH * EMBED_DIM,), jnp.float32),
    scratch_types=[
        pltpu.VMEM((_B_PER_W,), jnp.int32),
        pltpu.VMEM((_B_PER_W * EMBED_DIM,), jnp.int32),
        pltpu.VMEM((_B_PER_W * EMBED_DIM,), jnp.float32),
        pltpu.SemaphoreType.DMA,
    ],
    compiler_params=pltpu.CompilerParams(
        use_tc_tiling_on_sc=False, needs_layout_passes=False
    ),
)
def _sc_gather(tflat_hbm, idx_hbm, out_hbm, idx_v, fidx_v, rows_v, sem):
    wid = lax.axis_index("s") * _NC + lax.axis_index("c")
    base = wid * _B_PER_W
    pltpu.sync_copy(idx_hbm.at[pl.ds(base, _B_PER_W)], idx_v)
    lane = lax.iota(jnp.int32, 16)
    for h in range(_B_PER_W // 16):
        v = idx_v[pl.ds(h * 16, 16)]
        pos = (lane + h * 16) * EMBED_DIM
        for d in range(EMBED_DIM):
            plsc.store_scatter(fidx_v, [pos + d], v + d * VOCAB)
    pltpu.async_copy(tflat_hbm.at[fidx_v], rows_v, sem).wait()
    pltpu.sync_copy(rows_v, out_hbm.at[pl.ds(base * EMBED_DIM, _B_PER_W * EMBED_DIM)])


def _proj_body(wt_ref, e_ref, b_ref, o_ref):
    o_ref[...] = lax.dot_general(
        wt_ref[...],
        e_ref[...],
        dimension_numbers=(((0,), (1,)), ((), ())),
        preferred_element_type=jnp.float32,
    ) + b_ref[...].T


def _tc_project(embeds, lin_w, lin_b):
    out_t = pl.pallas_call(
        _proj_body,
        grid=(pl.cdiv(VOCAB, _TILE_M),),
        in_specs=[
            pl.BlockSpec((EMBED_DIM, _TILE_M), lambda m: (0, m)),
            pl.BlockSpec((BATCH, EMBED_DIM), lambda m: (0, 0)),
            pl.BlockSpec((1, _TILE_M), lambda m: (0, m)),
        ],
        out_specs=pl.BlockSpec((_TILE_M, BATCH), lambda m: (m, 0)),
        out_shape=jax.ShapeDtypeStruct((VOCAB, BATCH), jnp.float32),
    )(lin_w.T, embeds, lin_b.reshape(1, VOCAB))
    return out_t.T


def kernel(inputs, emb_table, lin_w, lin_b):
    tflat = emb_table.T.reshape(VOCAB * EMBED_DIM)
    embeds = _sc_gather(tflat, inputs.astype(jnp.int32)).reshape(BATCH, EMBED_DIM)
    return _tc_project(embeds, lin_w, lin_b)


# speedup vs baseline: 1.0101x; 1.0101x over previous
import functools

import jax
import jax.numpy as jnp
from jax import lax
from jax.experimental import pallas as pl
from jax.experimental.pallas import tpu as pltpu
from jax.experimental.pallas import tpu_sc as plsc

VOCAB = 100000
EMBED_DIM = 16
BATCH = 1024

_NC = 2
_NS = 16
_NW = _NC * _NS
_B_PER_W = BATCH // _NW

_TILE_M = 4096


@functools.partial(
    pl.kernel,
    mesh=plsc.VectorSubcoreMesh(core_axis_name="c", subcore_axis_name="s"),
    out_type=jax.ShapeDtypeStruct((BATCH * EMBED_DIM,), jnp.float32),
    scratch_types=[
        pltpu.VMEM((_B_PER_W,), jnp.int32),
        pltpu.VMEM((_B_PER_W * EMBED_DIM,), jnp.int32),
        pltpu.VMEM((_B_PER_W * EMBED_DIM,), jnp.float32),
        pltpu.SemaphoreType.DMA,
    ],
    compiler_params=pltpu.CompilerParams(
        use_tc_tiling_on_sc=False,
        needs_layout_passes=False,
        skip_device_barrier=True,
    ),
)
def _sc_gather(tflat_hbm, idx_hbm, out_hbm, idx_v, fidx_v, rows_v, sem):
    wid = lax.axis_index("s") * _NC + lax.axis_index("c")
    base = wid * _B_PER_W
    pltpu.sync_copy(idx_hbm.at[pl.ds(base, _B_PER_W)], idx_v)
    lane = lax.iota(jnp.int32, 16)
    for h in range(_B_PER_W // 16):
        v = idx_v[pl.ds(h * 16, 16)]
        pos = (lane + h * 16) * EMBED_DIM
        for d in range(EMBED_DIM):
            plsc.store_scatter(fidx_v, [pos + d], v + d * VOCAB)
    pltpu.async_copy(tflat_hbm.at[fidx_v], rows_v, sem).wait()
    pltpu.sync_copy(rows_v, out_hbm.at[pl.ds(base * EMBED_DIM, _B_PER_W * EMBED_DIM)])


def _proj_body(wt_ref, e_ref, b_ref, o_ref):
    o_ref[...] = lax.dot_general(
        wt_ref[...],
        e_ref[...],
        dimension_numbers=(((0,), (1,)), ((), ())),
        preferred_element_type=jnp.float32,
    ) + b_ref[...].T


def _tc_project(embeds, lin_w, lin_b):
    out_t = pl.pallas_call(
        _proj_body,
        grid=(pl.cdiv(VOCAB, _TILE_M),),
        in_specs=[
            pl.BlockSpec((EMBED_DIM, _TILE_M), lambda m: (0, m)),
            pl.BlockSpec((BATCH, EMBED_DIM), lambda m: (0, 0)),
            pl.BlockSpec((1, _TILE_M), lambda m: (0, m)),
        ],
        out_specs=pl.BlockSpec((_TILE_M, BATCH), lambda m: (m, 0)),
        out_shape=jax.ShapeDtypeStruct((VOCAB, BATCH), jnp.float32),
    )(lin_w.T, embeds, lin_b.reshape(1, VOCAB))
    return out_t.T


def kernel(inputs, emb_table, lin_w, lin_b):
    tflat = emb_table.T.reshape(VOCAB * EMBED_DIM)
    embeds = _sc_gather(tflat, inputs.astype(jnp.int32)).reshape(BATCH, EMBED_DIM)
    return _tc_project(embeds, lin_w, lin_b)


# speedup vs baseline: 1.0182x; 1.0080x over previous
import functools

import jax
import jax.numpy as jnp
from jax import lax
from jax.experimental import pallas as pl
from jax.experimental.pallas import tpu as pltpu
from jax.experimental.pallas import tpu_sc as plsc

VOCAB = 100000
EMBED_DIM = 16
BATCH = 1024

_NC = 1
_NS = 16
_NW = _NC * _NS
_B_PER_W = BATCH // _NW

_TILE_M = 4096


@functools.partial(
    pl.kernel,
    mesh=plsc.VectorSubcoreMesh(
        core_axis_name="c", subcore_axis_name="s", num_cores=_NC
    ),
    out_type=jax.ShapeDtypeStruct((BATCH * EMBED_DIM,), jnp.float32),
    scratch_types=[
        pltpu.VMEM((_B_PER_W,), jnp.int32),
        pltpu.VMEM((_B_PER_W * EMBED_DIM,), jnp.int32),
        pltpu.VMEM((_B_PER_W * EMBED_DIM,), jnp.float32),
        pltpu.SemaphoreType.DMA,
    ],
    compiler_params=pltpu.CompilerParams(
        use_tc_tiling_on_sc=False,
        needs_layout_passes=False,
    ),
)
def _sc_gather(tflat_hbm, idx_hbm, out_hbm, idx_v, fidx_v, rows_v, sem):
    wid = lax.axis_index("s") * _NC + lax.axis_index("c")
    base = wid * _B_PER_W
    pltpu.sync_copy(idx_hbm.at[pl.ds(base, _B_PER_W)], idx_v)
    lane = lax.iota(jnp.int32, 16)
    for h in range(_B_PER_W // 16):
        v = idx_v[pl.ds(h * 16, 16)]
        pos = (lane + h * 16) * EMBED_DIM
        for d in range(EMBED_DIM):
            plsc.store_scatter(fidx_v, [pos + d], v + d * VOCAB)
    pltpu.async_copy(tflat_hbm.at[fidx_v], rows_v, sem).wait()
    pltpu.sync_copy(rows_v, out_hbm.at[pl.ds(base * EMBED_DIM, _B_PER_W * EMBED_DIM)])


def _proj_body(wt_ref, e_ref, b_ref, o_ref):
    o_ref[...] = lax.dot_general(
        wt_ref[...],
        e_ref[...],
        dimension_numbers=(((0,), (1,)), ((), ())),
        preferred_element_type=jnp.float32,
    ) + b_ref[...].T


def _tc_project(embeds, lin_w, lin_b):
    out_t = pl.pallas_call(
        _proj_body,
        grid=(pl.cdiv(VOCAB, _TILE_M),),
        in_specs=[
            pl.BlockSpec((EMBED_DIM, _TILE_M), lambda m: (0, m)),
            pl.BlockSpec((BATCH, EMBED_DIM), lambda m: (0, 0)),
            pl.BlockSpec((1, _TILE_M), lambda m: (0, m)),
        ],
        out_specs=pl.BlockSpec((_TILE_M, BATCH), lambda m: (m, 0)),
        out_shape=jax.ShapeDtypeStruct((VOCAB, BATCH), jnp.float32),
    )(lin_w.T, embeds, lin_b.reshape(1, VOCAB))
    return out_t.T


def kernel(inputs, emb_table, lin_w, lin_b):
    tflat = emb_table.T.reshape(VOCAB * EMBED_DIM)
    embeds = _sc_gather(tflat, inputs.astype(jnp.int32)).reshape(BATCH, EMBED_DIM)
    return _tc_project(embeds, lin_w, lin_b)
